# baseline (speedup 1.0000x reference)
import functools

import jax
import jax.numpy as jnp
from jax import lax
from jax.experimental import pallas as pl
from jax.experimental.pallas import tpu as pltpu
from jax.experimental.pallas import tpu_sc as plsc

_NUM_CORES = 2
_NUM_SUBCORES = 16
_NUM_WORKERS = _NUM_CORES * _NUM_SUBCORES
_BW = 128
_LANES = 16


@functools.lru_cache(maxsize=None)
def _make_lookup(s: int, bsz: int, dim: int):
    assert bsz == _BW * _NUM_WORKERS
    mesh = plsc.VectorSubcoreMesh(core_axis_name="c", subcore_axis_name="s")

    @functools.partial(
        pl.kernel,
        out_type=jax.ShapeDtypeStruct((s, dim, bsz), jnp.float32),
        mesh=mesh,
        scratch_types=[
            pltpu.VMEM((s, _BW), jnp.int32),
            pltpu.VMEM((2, _BW, dim), jnp.float32),
            pltpu.VMEM((2, dim, _BW), jnp.float32),
            pltpu.SemaphoreType.DMA,
        ]
        + [pltpu.SemaphoreType.DMA] * 2
        + [pltpu.SemaphoreType.DMA] * 2,
        compiler_params=pltpu.CompilerParams(
            use_tc_tiling_on_sc=False, needs_layout_passes=False),
    )
    def lookup(idx_hbm, table_hbm, out_hbm, idx_v, rows_v, tr_v, isem,
               gsem0, gsem1, osem0, osem1):
        gsem = (gsem0, gsem1)
        osem = (osem0, osem1)
        wid = lax.axis_index("s") * _NUM_CORES + lax.axis_index("c")
        b0 = wid * _BW

        def fetch(t, p):
            return pltpu.async_copy(
                table_hbm.at[idx_v.at[t]], rows_v.at[p], gsem[p])

        def transpose(p):
            rowsp = rows_v.at[p]
            trp = tr_v.at[p]
            row_ids = [
                jax.lax.iota(jnp.int32, _LANES) + g * _LANES
                for g in range(_BW // _LANES)
            ]

            def per_d(d, carry):
                col = jnp.full((_LANES,), d, jnp.int32)
                for g in range(_BW // _LANES):
                    trp[d, pl.ds(g * _LANES, _LANES)] = plsc.load_gather(
                        rowsp, [row_ids[g], col])
                return carry

            lax.fori_loop(0, dim, per_d, None)

        def put_out(t, p):
            return pltpu.async_copy(
                tr_v.at[p], out_hbm.at[t, :, pl.ds(b0, _BW)], osem[p])

        def drain_out(p):
            pltpu.make_async_copy(
                tr_v.at[p], out_hbm.at[0, :, pl.ds(b0, _BW)], osem[p]).wait()

        def drain_gather(p):
            pltpu.make_async_copy(
                table_hbm.at[idx_v.at[0]], rows_v.at[p], gsem[p]).wait()

        pltpu.async_copy(
            idx_hbm.at[:, pl.ds(b0, _BW)], idx_v, isem).wait()

        fetch(0, 0)
        fetch(1, 1)
        for t in (0, 1):
            p = t & 1
            drain_gather(p)
            transpose(p)
            fetch(t + 2, p)
            put_out(t, p)

        def group(g, carry):
            for p in (0, 1):
                t = 2 * g + p
                drain_out(p)
                drain_gather(p)
                transpose(p)
                fetch(lax.min(t + 2, s - 1), p)
                put_out(t, p)
            return carry

        lax.fori_loop(1, s // 2, group, None)

        for p in (0, 1):
            drain_gather(p)
            drain_out(p)

    return lookup


def kernel(token_ids, weight):
    b, s = token_ids.shape
    dim = weight.shape[1]
    ids_t = token_ids.T.astype(jnp.int32)
    o = _make_lookup(s, b, dim)(ids_t, weight)
    return o.transpose(2, 0, 1)


# speedup vs baseline: 1.4768x; 1.4768x over previous
import functools

import jax
import jax.numpy as jnp
from jax import lax
from jax.experimental import pallas as pl
from jax.experimental.pallas import tpu as pltpu
from jax.experimental.pallas import tpu_sc as plsc

_NUM_CORES = 2
_NUM_SUBCORES = 16
_NUM_WORKERS = _NUM_CORES * _NUM_SUBCORES
_BW = 128
_LANES = 16


@functools.lru_cache(maxsize=None)
def _make_lookup(s: int, bsz: int, dim: int):
    assert bsz == _BW * _NUM_WORKERS
    mesh = plsc.VectorSubcoreMesh(core_axis_name="c", subcore_axis_name="s")

    @functools.partial(
        pl.kernel,
        out_type=jax.ShapeDtypeStruct((s, dim, bsz), jnp.float32),
        mesh=mesh,
        scratch_types=[
            pltpu.VMEM((s, _BW), jnp.int32),
            pltpu.VMEM((2, _BW, dim), jnp.float32),
            pltpu.VMEM((2, dim, _BW + 1), jnp.float32),
            pltpu.SemaphoreType.DMA,
        ]
        + [pltpu.SemaphoreType.DMA] * 2
        + [pltpu.SemaphoreType.DMA] * 2,
        compiler_params=pltpu.CompilerParams(
            use_tc_tiling_on_sc=False, needs_layout_passes=False),
    )
    def lookup(idx_hbm, table_hbm, out_hbm, idx_v, rows_v, tr_v, isem,
               gsem0, gsem1, osem0, osem1):
        gsem = (gsem0, gsem1)
        osem = (osem0, osem1)
        wid = lax.axis_index("s") * _NUM_CORES + lax.axis_index("c")
        b0 = wid * _BW

        def fetch(t, p):
            return pltpu.async_copy(
                table_hbm.at[idx_v.at[t]], rows_v.at[p], gsem[p])

        def transpose(p):
            rowsp = rows_v.at[p]
            trp = tr_v.at[p]
            iota = jax.lax.iota(jnp.int32, _LANES)
            for d0 in range(0, dim, _LANES):
                d_ids = iota + d0
                for b in range(_BW):
                    plsc.store_scatter(
                        trp, [d_ids, jnp.full((_LANES,), b, jnp.int32)],
                        rowsp[b, pl.ds(d0, _LANES)])

        def put_out(t, p):
            return pltpu.async_copy(
                tr_v.at[p, :, pl.ds(0, _BW)],
                out_hbm.at[t, :, pl.ds(b0, _BW)], osem[p])

        def drain_out(p):
            pltpu.make_async_copy(
                tr_v.at[p, :, pl.ds(0, _BW)],
                out_hbm.at[0, :, pl.ds(b0, _BW)], osem[p]).wait()

        def drain_gather(p):
            pltpu.make_async_copy(
                table_hbm.at[idx_v.at[0]], rows_v.at[p], gsem[p]).wait()

        pltpu.async_copy(
            idx_hbm.at[:, pl.ds(b0, _BW)], idx_v, isem).wait()

        fetch(0, 0)
        fetch(1, 1)
        for t in (0, 1):
            p = t & 1
            drain_gather(p)
            transpose(p)
            fetch(t + 2, p)
            put_out(t, p)

        def group(g, carry):
            for p in (0, 1):
                t = 2 * g + p
                drain_out(p)
                drain_gather(p)
                transpose(p)
                fetch(lax.min(t + 2, s - 1), p)
                put_out(t, p)
            return carry

        lax.fori_loop(1, s // 2, group, None)

        for p in (0, 1):
            drain_gather(p)
            drain_out(p)

    return lookup


def kernel(token_ids, weight):
    b, s = token_ids.shape
    dim = weight.shape[1]
    ids_t = token_ids.T.astype(jnp.int32)
    o = _make_lookup(s, b, dim)(ids_t, weight)
    return o.transpose(2, 0, 1)


# speedup vs baseline: 1.6400x; 1.1105x over previous
import functools

import jax
import jax.numpy as jnp
from jax import lax
from jax.experimental import pallas as pl
from jax.experimental.pallas import tpu as pltpu
from jax.experimental.pallas import tpu_sc as plsc

_NUM_CORES = 2
_NUM_SUBCORES = 16
_NUM_WORKERS = _NUM_CORES * _NUM_SUBCORES
_BW = 128
_LANES = 16


@functools.lru_cache(maxsize=None)
def _make_lookup(s: int, bsz: int, dim: int):
    assert bsz == _BW * _NUM_WORKERS
    mesh = plsc.VectorSubcoreMesh(core_axis_name="c", subcore_axis_name="s")

    @functools.partial(
        pl.kernel,
        out_type=jax.ShapeDtypeStruct((s, dim, bsz), jnp.float32),
        mesh=mesh,
        scratch_types=[
            pltpu.VMEM((s, _BW), jnp.int32),
            pltpu.VMEM((2, _BW, dim), jnp.float32),
            pltpu.VMEM((2, dim, _BW + 1), jnp.float32),
            pltpu.SemaphoreType.DMA,
        ]
        + [pltpu.SemaphoreType.DMA] * 2
        + [pltpu.SemaphoreType.DMA] * 2,
        compiler_params=pltpu.CompilerParams(
            use_tc_tiling_on_sc=False, needs_layout_passes=False),
    )
    def lookup(idx_hbm, table_hbm, out_hbm, idx_v, rows_v, tr_v, isem,
               gsem0, gsem1, osem0, osem1):
        gsem = (gsem0, gsem1)
        osem = (osem0, osem1)
        wid = lax.axis_index("s") * _NUM_CORES + lax.axis_index("c")
        b0 = wid * _BW

        def fetch(t, p):
            return pltpu.async_copy(
                table_hbm.at[idx_v.at[t]], rows_v.at[p], gsem[p])

        iota = jax.lax.iota(jnp.int32, _LANES)
        d_ids = [iota + d0 for d0 in range(0, dim, _LANES)]
        _BUNROLL = 4

        def transpose(p):
            rowsp = rows_v.at[p]
            trp = tr_v.at[p]

            def body(g, carry):
                for k in range(_BUNROLL):
                    b = g * _BUNROLL + k
                    bvec = jnp.full((_LANES,), 0, jnp.int32) + b
                    for j in range(dim // _LANES):
                        plsc.store_scatter(
                            trp, [d_ids[j], bvec],
                            rowsp[b, pl.ds(j * _LANES, _LANES)])
                return carry

            lax.fori_loop(0, _BW // _BUNROLL, body, None)

        def put_out(t, p):
            return pltpu.async_copy(
                tr_v.at[p, :, pl.ds(0, _BW)],
                out_hbm.at[t, :, pl.ds(b0, _BW)], osem[p])

        def drain_out(p):
            pltpu.make_async_copy(
                tr_v.at[p, :, pl.ds(0, _BW)],
                out_hbm.at[0, :, pl.ds(b0, _BW)], osem[p]).wait()

        def drain_gather(p):
            pltpu.make_async_copy(
                table_hbm.at[idx_v.at[0]], rows_v.at[p], gsem[p]).wait()

        pltpu.async_copy(
            idx_hbm.at[:, pl.ds(b0, _BW)], idx_v, isem).wait()

        fetch(0, 0)
        fetch(1, 1)
        for t in (0, 1):
            p = t & 1
            drain_gather(p)
            transpose(p)
            fetch(t + 2, p)
            put_out(t, p)

        def group(g, carry):
            for p in (0, 1):
                t = 2 * g + p
                drain_out(p)
                drain_gather(p)
                transpose(p)
                fetch(lax.min(t + 2, s - 1), p)
                put_out(t, p)
            return carry

        lax.fori_loop(1, s // 2, group, None)

        for p in (0, 1):
            drain_gather(p)
            drain_out(p)

    return lookup


def kernel(token_ids, weight):
    b, s = token_ids.shape
    dim = weight.shape[1]
    ids_t = token_ids.T.astype(jnp.int32)
    o = _make_lookup(s, b, dim)(ids_t, weight)
    return o.transpose(2, 0, 1)


# speedup vs baseline: 1.7362x; 1.0587x over previous
import functools

import jax
import jax.numpy as jnp
from jax import lax
from jax.experimental import pallas as pl
from jax.experimental.pallas import tpu as pltpu
from jax.experimental.pallas import tpu_sc as plsc

_NUM_CORES = 2
_NUM_SUBCORES = 16
_NUM_WORKERS = _NUM_CORES * _NUM_SUBCORES
_BW = 128
_LANES = 16


@functools.lru_cache(maxsize=None)
def _make_lookup(s: int, bsz: int, dim: int):
    assert bsz == _BW * _NUM_WORKERS
    mesh = plsc.VectorSubcoreMesh(core_axis_name="c", subcore_axis_name="s")

    @functools.partial(
        pl.kernel,
        out_type=jax.ShapeDtypeStruct((s, dim, bsz), jnp.float32),
        mesh=mesh,
        scratch_types=[
            pltpu.VMEM((s, _BW), jnp.int32),
            pltpu.VMEM((2, _BW, 128), jnp.float32),
            pltpu.VMEM((2, dim, _BW + 1), jnp.float32),
            pltpu.SemaphoreType.DMA,
        ]
        + [pltpu.SemaphoreType.DMA] * 2
        + [pltpu.SemaphoreType.DMA] * 2,
        compiler_params=pltpu.CompilerParams(
            use_tc_tiling_on_sc=False, needs_layout_passes=False),
    )
    def lookup(idx_hbm, table_hbm, out_hbm, idx_v, rows_v, tr_v, isem,
               gsem0, gsem1, osem0, osem1):
        gsem = (gsem0, gsem1)
        osem = (osem0, osem1)
        wid = lax.axis_index("s") * _NUM_CORES + lax.axis_index("c")
        b0 = wid * _BW

        def fetch(t, p):
            return pltpu.async_copy(
                table_hbm.at[idx_v.at[t]], rows_v.at[p], gsem[p])

        iota = jax.lax.iota(jnp.int32, _LANES)
        d_ids = [iota + d0 for d0 in range(0, dim, _LANES)]
        _BUNROLL = 4

        def transpose(p):
            rowsp = rows_v.at[p]
            trp = tr_v.at[p]

            def body(g, carry):
                for k in range(_BUNROLL):
                    b = g * _BUNROLL + k
                    bvec = jnp.full((_LANES,), 0, jnp.int32) + b
                    for j in range(dim // _LANES):
                        plsc.store_scatter(
                            trp, [d_ids[j], bvec],
                            rowsp[b, pl.ds(j * _LANES, _LANES)])
                return carry

            lax.fori_loop(0, _BW // _BUNROLL, body, None)

        def put_out(t, p):
            return pltpu.async_copy(
                tr_v.at[p, :, pl.ds(0, _BW)],
                out_hbm.at[t, :, pl.ds(b0, _BW)], osem[p])

        def drain_out(p):
            pltpu.make_async_copy(
                tr_v.at[p, :, pl.ds(0, _BW)],
                out_hbm.at[0, :, pl.ds(b0, _BW)], osem[p]).wait()

        def drain_gather(p):
            pltpu.make_async_copy(
                table_hbm.at[idx_v.at[0]], rows_v.at[p], gsem[p]).wait()

        pltpu.async_copy(
            idx_hbm.at[:, pl.ds(b0, _BW)], idx_v, isem).wait()

        fetch(0, 0)
        fetch(1, 1)
        for t in (0, 1):
            p = t & 1
            drain_gather(p)
            transpose(p)
            fetch(t + 2, p)
            put_out(t, p)

        def group(g, carry):
            for p in (0, 1):
                t = 2 * g + p
                drain_out(p)
                drain_gather(p)
                transpose(p)
                fetch(lax.min(t + 2, s - 1), p)
                put_out(t, p)
            return carry

        lax.fori_loop(1, s // 2, group, None)

        for p in (0, 1):
            drain_gather(p)
            drain_out(p)

    return lookup


def kernel(token_ids, weight):
    b, s = token_ids.shape
    dim = weight.shape[1]
    ids_t = token_ids.T.astype(jnp.int32)
    wpad = jnp.pad(weight, ((0, 0), (0, 128 - dim)))
    o = _make_lookup(s, b, dim)(ids_t, wpad)
    return o.transpose(2, 0, 1)


# speedup vs baseline: 2.1834x; 1.2576x over previous
import functools

import jax
import jax.numpy as jnp
from jax import lax
from jax.experimental import pallas as pl
from jax.experimental.pallas import tpu as pltpu
from jax.experimental.pallas import tpu_sc as plsc

_NUM_CORES = 2
_NUM_SUBCORES = 16
_NUM_WORKERS = _NUM_CORES * _NUM_SUBCORES
_BW = 128
_LANES = 16


@functools.lru_cache(maxsize=None)
def _make_lookup(s: int, bsz: int, dim: int):
    assert bsz == _BW * _NUM_WORKERS
    mesh = plsc.VectorSubcoreMesh(core_axis_name="c", subcore_axis_name="s")

    @functools.partial(
        pl.kernel,
        out_type=jax.ShapeDtypeStruct(
            (s, dim // 8, bsz // _BW, 8, _BW), jnp.float32),
        mesh=mesh,
        scratch_types=[
            pltpu.VMEM((s, _BW), jnp.int32),
            pltpu.VMEM((2, _BW, 128), jnp.float32),
            pltpu.VMEM((2, dim, _BW + 1), jnp.float32),
            pltpu.SemaphoreType.DMA,
        ]
        + [pltpu.SemaphoreType.DMA] * 2
        + [pltpu.SemaphoreType.DMA] * 2,
        compiler_params=pltpu.CompilerParams(
            use_tc_tiling_on_sc=False, needs_layout_passes=False),
    )
    def lookup(idx_hbm, table_hbm, out_hbm, idx_v, rows_v, tr_v, isem,
               gsem0, gsem1, osem0, osem1):
        gsem = (gsem0, gsem1)
        osem = (osem0, osem1)
        wid = lax.axis_index("s") * _NUM_CORES + lax.axis_index("c")
        b0 = wid * _BW

        def fetch(t, p):
            return pltpu.async_copy(
                table_hbm.at[idx_v.at[t]], rows_v.at[p], gsem[p])

        iota = jax.lax.iota(jnp.int32, _LANES)
        d_ids = [iota + d0 for d0 in range(0, dim, _LANES)]
        _BUNROLL = 4

        def transpose(p):
            rowsp = rows_v.at[p]
            trp = tr_v.at[p]

            def body(g, carry):
                for k in range(_BUNROLL):
                    b = g * _BUNROLL + k
                    bvec = jnp.full((_LANES,), 0, jnp.int32) + b
                    for j in range(dim // _LANES):
                        plsc.store_scatter(
                            trp, [d_ids[j], bvec],
                            rowsp[b, pl.ds(j * _LANES, _LANES)])
                return carry

            lax.fori_loop(0, _BW // _BUNROLL, body, None)

        def put_out(t, p):
            for dh in range(dim // 8):
                pltpu.async_copy(
                    tr_v.at[p, pl.ds(dh * 8, 8), pl.ds(0, _BW)],
                    out_hbm.at[t, dh, wid], osem[p])

        def drain_out(p):
            for dh in range(dim // 8):
                pltpu.make_async_copy(
                    tr_v.at[p, pl.ds(dh * 8, 8), pl.ds(0, _BW)],
                    out_hbm.at[0, dh, wid], osem[p]).wait()

        def drain_gather(p):
            pltpu.make_async_copy(
                table_hbm.at[idx_v.at[0]], rows_v.at[p], gsem[p]).wait()

        pltpu.async_copy(
            idx_hbm.at[:, pl.ds(b0, _BW)], idx_v, isem).wait()

        fetch(0, 0)
        fetch(1, 1)
        for t in (0, 1):
            p = t & 1
            drain_gather(p)
            transpose(p)
            fetch(t + 2, p)
            put_out(t, p)

        def group(g, carry):
            for p in (0, 1):
                t = 2 * g + p
                drain_out(p)
                drain_gather(p)
                transpose(p)
                fetch(lax.min(t + 2, s - 1), p)
                put_out(t, p)
            return carry

        lax.fori_loop(1, s // 2, group, None)

        for p in (0, 1):
            drain_gather(p)
            drain_out(p)

    return lookup


def kernel(token_ids, weight):
    b, s = token_ids.shape
    dim = weight.shape[1]
    ids_t = token_ids.T.astype(jnp.int32)
    wpad = jnp.pad(weight, ((0, 0), (0, 128 - dim)))
    o5 = _make_lookup(s, b, dim)(ids_t, wpad)
    return o5.transpose(2, 4, 0, 1, 3).reshape(b, s, dim)


# speedup vs baseline: 2.3581x; 1.0800x over previous
import functools

import jax
import jax.numpy as jnp
from jax import lax
from jax.experimental import pallas as pl
from jax.experimental.pallas import tpu as pltpu
from jax.experimental.pallas import tpu_sc as plsc

_NUM_CORES = 2
_NUM_SUBCORES = 16
_NUM_WORKERS = _NUM_CORES * _NUM_SUBCORES
_BW = 128
_LANES = 16


@functools.lru_cache(maxsize=None)
def _make_lookup(s: int, bsz: int, dim: int):
    assert bsz == _BW * _NUM_WORKERS
    mesh = plsc.VectorSubcoreMesh(core_axis_name="c", subcore_axis_name="s")

    @functools.partial(
        pl.kernel,
        out_type=jax.ShapeDtypeStruct(
            (s, dim // 8, bsz // _BW, 8, _BW), jnp.float32),
        mesh=mesh,
        scratch_types=[
            pltpu.VMEM((s, _BW), jnp.int32),
            pltpu.VMEM((2, _BW, 128), jnp.float32),
            pltpu.VMEM((2, dim, _BW + 1), jnp.float32),
            pltpu.SemaphoreType.DMA,
        ]
        + [pltpu.SemaphoreType.DMA] * 2
        + [pltpu.SemaphoreType.DMA] * 2,
        compiler_params=pltpu.CompilerParams(
            use_tc_tiling_on_sc=False, needs_layout_passes=False),
    )
    def lookup(idx_hbm, table_hbm, out_hbm, idx_v, rows_v, tr_v, isem,
               gsem0, gsem1, osem0, osem1):
        gsem = (gsem0, gsem1)
        osem = (osem0, osem1)
        wid = lax.axis_index("s") * _NUM_CORES + lax.axis_index("c")
        b0 = wid * _BW

        def fetch(t, p):
            return pltpu.async_copy(
                table_hbm.at[idx_v.at[t]], rows_v.at[p], gsem[p])

        iota = jax.lax.iota(jnp.int32, _LANES)
        d_ids = [iota + d0 for d0 in range(0, dim, _LANES)]
        _BUNROLL = 4

        def transpose(p):
            rowsp = rows_v.at[p]
            trp = tr_v.at[p]

            def body(g, carry):
                for k in range(_BUNROLL):
                    b = g * _BUNROLL + k
                    bvec = jnp.full((_LANES,), 0, jnp.int32) + b
                    for j in range(dim // _LANES):
                        plsc.store_scatter(
                            trp, [d_ids[j], bvec],
                            rowsp[b, pl.ds(j * _LANES, _LANES)])
                return carry

            lax.fori_loop(0, _BW // _BUNROLL, body, None)

        def put_out(t, p):
            for dh in range(dim // 8):
                pltpu.async_copy(
                    tr_v.at[p, pl.ds(dh * 8, 8), pl.ds(0, _BW)],
                    out_hbm.at[t, dh, wid], osem[p])

        def drain_out(p):
            for dh in range(dim // 8):
                pltpu.make_async_copy(
                    tr_v.at[p, pl.ds(dh * 8, 8), pl.ds(0, _BW)],
                    out_hbm.at[0, dh, wid], osem[p]).wait()

        def drain_gather(p):
            pltpu.make_async_copy(
                table_hbm.at[idx_v.at[0]], rows_v.at[p], gsem[p]).wait()

        pltpu.async_copy(
            idx_hbm.at[:, pl.ds(b0, _BW)], idx_v, isem).wait()

        fetch(0, 0)
        fetch(1, 1)
        for t in (0, 1):
            p = t & 1
            drain_gather(p)
            transpose(p)
            fetch(t + 2, p)
            put_out(t, p)

        def group(g, carry):
            for p in (0, 1):
                t = 2 * g + p
                drain_out(p)
                drain_gather(p)
                transpose(p)
                fetch(lax.min(t + 2, s - 1), p)
                put_out(t, p)
            return carry

        lax.fori_loop(1, s // 2, group, None)

        for p in (0, 1):
            drain_gather(p)
            drain_out(p)

    return lookup


_TNB = 2048


@functools.lru_cache(maxsize=None)
def _make_prep(n: int, dim: int):
    def body(x_ref, o_ref):
        o_ref[:, :dim] = x_ref[...].T
        o_ref[:, dim:] = jnp.zeros((_TNB, 128 - dim), jnp.float32)

    return pl.pallas_call(
        body,
        grid=(pl.cdiv(n, _TNB),),
        in_specs=[pl.BlockSpec((dim, _TNB), lambda i: (0, i))],
        out_specs=pl.BlockSpec((_TNB, 128), lambda i: (i, 0)),
        out_shape=jax.ShapeDtypeStruct((n, 128), jnp.float32),
    )


def kernel(token_ids, weight):
    b, s = token_ids.shape
    dim = weight.shape[1]
    ids_t = token_ids.T.astype(jnp.int32)
    wpad = _make_prep(weight.shape[0], dim)(weight.T)
    o5 = _make_lookup(s, b, dim)(ids_t, wpad)
    return o5.transpose(2, 4, 0, 1, 3).reshape(b, s, dim)


# speedup vs baseline: 3.0616x; 1.2983x over previous
import functools

import jax
import jax.numpy as jnp
from jax import lax
from jax.experimental import pallas as pl
from jax.experimental.pallas import tpu as pltpu
from jax.experimental.pallas import tpu_sc as plsc

_NUM_CORES = 2
_NUM_SUBCORES = 16
_NUM_WORKERS = _NUM_CORES * _NUM_SUBCORES
_BW = 128
_LANES = 16


@functools.lru_cache(maxsize=None)
def _make_lookup(s: int, bsz: int, dim: int):
    assert bsz == _BW * _NUM_WORKERS
    mesh = plsc.VectorSubcoreMesh(core_axis_name="c", subcore_axis_name="s")

    @functools.partial(
        pl.kernel,
        out_type=jax.ShapeDtypeStruct(
            (s, dim // 8, bsz // _BW, 8, _BW), jnp.float32),
        mesh=mesh,
        scratch_types=[
            pltpu.VMEM((s, _BW), jnp.int32),
            pltpu.VMEM((2, _BW, 128), jnp.float32),
            pltpu.VMEM((2, dim, _BW + 1), jnp.float32),
            pltpu.SemaphoreType.DMA,
        ]
        + [pltpu.SemaphoreType.DMA] * 2
        + [pltpu.SemaphoreType.DMA] * 2,
        compiler_params=pltpu.CompilerParams(
            use_tc_tiling_on_sc=False, needs_layout_passes=False),
    )
    def lookup(idx_hbm, table_hbm, out_hbm, idx_v, rows_v, tr_v, isem,
               gsem0, gsem1, osem0, osem1):
        gsem = (gsem0, gsem1)
        osem = (osem0, osem1)
        wid = lax.axis_index("s") * _NUM_CORES + lax.axis_index("c")
        b0 = wid * _BW

        def fetch(t, p):
            return pltpu.async_copy(
                table_hbm.at[idx_v.at[t]], rows_v.at[p], gsem[p])

        iota = jax.lax.iota(jnp.int32, _LANES)
        d_ids = [iota + d0 for d0 in range(0, dim, _LANES)]
        _BUNROLL = 8

        def transpose(p):
            rowsp = rows_v.at[p]
            trp = tr_v.at[p]

            def body(g, carry):
                for k in range(_BUNROLL):
                    b = g * _BUNROLL + k
                    bvec = jnp.full((_LANES,), 0, jnp.int32) + b
                    for j in range(dim // _LANES):
                        plsc.store_scatter(
                            trp, [d_ids[j], bvec],
                            rowsp[b, pl.ds(j * _LANES, _LANES)])
                return carry

            lax.fori_loop(0, _BW // _BUNROLL, body, None)

        def put_out(t, p):
            for dh in range(dim // 8):
                pltpu.async_copy(
                    tr_v.at[p, pl.ds(dh * 8, 8), pl.ds(0, _BW)],
                    out_hbm.at[t, dh, wid], osem[p])

        def drain_out(p):
            for dh in range(dim // 8):
                pltpu.make_async_copy(
                    tr_v.at[p, pl.ds(dh * 8, 8), pl.ds(0, _BW)],
                    out_hbm.at[0, dh, wid], osem[p]).wait()

        def drain_gather(p):
            pltpu.make_async_copy(
                table_hbm.at[idx_v.at[0]], rows_v.at[p], gsem[p]).wait()

        pltpu.async_copy(
            idx_hbm.at[:, pl.ds(b0, _BW)], idx_v, isem).wait()

        fetch(0, 0)
        fetch(1, 1)
        for t in (0, 1):
            p = t & 1
            drain_gather(p)
            transpose(p)
            fetch(t + 2, p)
            put_out(t, p)

        def group(g, carry):
            for p in (0, 1):
                t = 2 * g + p
                drain_out(p)
                drain_gather(p)
                transpose(p)
                fetch(lax.min(t + 2, s - 1), p)
                put_out(t, p)
            return carry

        lax.fori_loop(1, s // 2, group, None)

        for p in (0, 1):
            drain_gather(p)
            drain_out(p)

    return lookup


_TNB = 8192


@functools.lru_cache(maxsize=None)
def _make_prep(n: int, dim: int):
    def body(x_ref, o_ref):
        o_ref[:, :dim] = x_ref[...].T
        o_ref[:, dim:] = jnp.zeros((_TNB, 128 - dim), jnp.float32)

    return pl.pallas_call(
        body,
        grid=(pl.cdiv(n, _TNB),),
        in_specs=[pl.BlockSpec((dim, _TNB), lambda i: (0, i))],
        out_specs=pl.BlockSpec((_TNB, 128), lambda i: (i, 0)),
        out_shape=jax.ShapeDtypeStruct((n, 128), jnp.float32),
    )


def kernel(token_ids, weight):
    b, s = token_ids.shape
    dim = weight.shape[1]
    ids_t = token_ids.T.astype(jnp.int32)
    wpad = _make_prep(weight.shape[0], dim)(weight.T)
    o5 = _make_lookup(s, b, dim)(ids_t, wpad)
    return o5.transpose(2, 4, 0, 1, 3).reshape(b, s, dim)


# speedup vs baseline: 3.1562x; 1.0309x over previous
import functools

import jax
import jax.numpy as jnp
from jax import lax
from jax.experimental import pallas as pl
from jax.experimental.pallas import tpu as pltpu
from jax.experimental.pallas import tpu_sc as plsc

_NUM_CORES = 2
_NUM_SUBCORES = 16
_NUM_WORKERS = _NUM_CORES * _NUM_SUBCORES
_BW = 128
_LANES = 16


@functools.lru_cache(maxsize=None)
def _make_lookup(s: int, bsz: int, dim: int):
    assert bsz == _BW * _NUM_WORKERS
    mesh = plsc.VectorSubcoreMesh(core_axis_name="c", subcore_axis_name="s")

    @functools.partial(
        pl.kernel,
        out_type=jax.ShapeDtypeStruct(
            (s, dim // 8, bsz // _BW, 8, _BW), jnp.float32),
        mesh=mesh,
        scratch_types=[
            pltpu.VMEM((s, _BW), jnp.int32),
            pltpu.VMEM((2, _BW, 128), jnp.float32),
            pltpu.VMEM((2, dim, _BW + 1), jnp.float32),
            pltpu.SemaphoreType.DMA,
        ]
        + [pltpu.SemaphoreType.DMA] * 2
        + [pltpu.SemaphoreType.DMA] * 2,
        compiler_params=pltpu.CompilerParams(
            use_tc_tiling_on_sc=False, needs_layout_passes=False),
    )
    def lookup(idx_hbm, table_hbm, out_hbm, idx_v, rows_v, tr_v, isem,
               gsem0, gsem1, osem0, osem1):
        gsem = (gsem0, gsem1)
        osem = (osem0, osem1)
        wid = lax.axis_index("s") * _NUM_CORES + lax.axis_index("c")
        b0 = wid * _BW

        def fetch(t, p):
            return pltpu.async_copy(
                table_hbm.at[idx_v.at[t]], rows_v.at[p], gsem[p])

        iota = jax.lax.iota(jnp.int32, _LANES)
        d_ids = [iota + d0 for d0 in range(0, dim, _LANES)]
        _BUNROLL = 8

        def transpose(p):
            rowsp = rows_v.at[p]
            trp = tr_v.at[p]

            def body(g, carry):
                for k in range(_BUNROLL):
                    b = g * _BUNROLL + k
                    bvec = jnp.full((_LANES,), 0, jnp.int32) + b
                    for j in range(dim // _LANES):
                        plsc.store_scatter(
                            trp, [d_ids[j], bvec],
                            rowsp[b, pl.ds(j * _LANES, _LANES)])
                return carry

            lax.fori_loop(0, _BW // _BUNROLL, body, None)

        def put_out(t, p):
            for dh in range(dim // 8):
                pltpu.async_copy(
                    tr_v.at[p, pl.ds(dh * 8, 8), pl.ds(0, _BW)],
                    out_hbm.at[t, dh, wid], osem[p])

        def drain_out(p):
            for dh in range(dim // 8):
                pltpu.make_async_copy(
                    tr_v.at[p, pl.ds(dh * 8, 8), pl.ds(0, _BW)],
                    out_hbm.at[0, dh, wid], osem[p]).wait()

        def drain_gather(p):
            pltpu.make_async_copy(
                table_hbm.at[idx_v.at[0]], rows_v.at[p], gsem[p]).wait()

        pltpu.async_copy(
            idx_hbm.at[:, pl.ds(b0, _BW)], idx_v, isem).wait()

        fetch(0, 0)
        fetch(1, 1)
        for t in (0, 1):
            p = t & 1
            drain_gather(p)
            transpose(p)
            fetch(t + 2, p)
            put_out(t, p)

        def group(g, carry):
            for p in (0, 1):
                t = 2 * g + p
                drain_out(p)
                drain_gather(p)
                transpose(p)
                fetch(lax.min(t + 2, s - 1), p)
                put_out(t, p)
            return carry

        lax.fori_loop(1, s // 2, group, None)

        for p in (0, 1):
            drain_gather(p)
            drain_out(p)

    return lookup


_TNB = 16384


@functools.lru_cache(maxsize=None)
def _make_prep(n: int, dim: int):
    def body(x_ref, o_ref):
        o_ref[:, :dim] = x_ref[...].T
        o_ref[:, dim:] = jnp.zeros((_TNB, 128 - dim), jnp.float32)

    return pl.pallas_call(
        body,
        grid=(pl.cdiv(n, _TNB),),
        in_specs=[pl.BlockSpec((dim, _TNB), lambda i: (0, i))],
        out_specs=pl.BlockSpec((_TNB, 128), lambda i: (i, 0)),
        out_shape=jax.ShapeDtypeStruct((n, 128), jnp.float32),
    )


def kernel(token_ids, weight):
    b, s = token_ids.shape
    dim = weight.shape[1]
    ids_t = token_ids.T.astype(jnp.int32)
    wpad = _make_prep(weight.shape[0], dim)(weight.T)
    o5 = _make_lookup(s, b, dim)(ids_t, wpad)
    return o5.transpose(2, 4, 0, 1, 3).reshape(b, s, dim)
